# baseline (speedup 1.0000x reference)
import jax
import jax.numpy as jnp
from jax import lax
from jax.experimental import pallas as pl
from jax.experimental.pallas import tpu as pltpu

D_MODEL = 256
SINK_ITERS = 50
N = 1000
PAD = 1024


def _ot_kernel(xlc_ref, xte_ref, w1l_ref, b1l_ref, w2l_ref, b2l_ref,
               w1t_ref, b1t_ref, w2t_ref, b2t_ref, alpha_ref, out_ref):
    f32 = jnp.float32
    xlc = xlc_ref[0]
    xte = xte_ref[0]

    h = jnp.maximum(jnp.dot(xlc, w1l_ref[...], preferred_element_type=f32)
                    + b1l_ref[...], 0.0)
    f_lc = jnp.dot(h, w2l_ref[...], preferred_element_type=f32) + b2l_ref[...]
    h = jnp.maximum(jnp.dot(xte, w1t_ref[...], preferred_element_type=f32)
                    + b1t_ref[...], 0.0)
    f_te = jnp.dot(h, w2t_ref[...], preferred_element_type=f32) + b2t_ref[...]

    inv_sqrt_d = 1.0 / (D_MODEL ** 0.5)
    bf = jnp.bfloat16
    f_lc_b = f_lc.astype(bf)
    f_te_b = f_te.astype(bf)
    s = lax.dot_general(f_lc_b, f_te_b, (((1,), (1,)), ((), ())),
                        preferred_element_type=f32) * inv_sqrt_d
    st = lax.dot_general(f_te_b, f_lc_b, (((1,), (1,)), ((), ())),
                         preferred_element_type=f32) * inv_sqrt_d

    alpha = alpha_ref[0, 0]
    ri = lax.broadcasted_iota(jnp.int32, (PAD, PAD), 0)
    ci = lax.broadcasted_iota(jnp.int32, (PAD, PAD), 1)
    in_scores = (ri < N) & (ci < N)
    in_coupl = (ri <= N) & (ci <= N)

    C = jnp.where(in_scores, s, alpha)
    K = jnp.where(in_coupl, jnp.exp(C), 0.0).astype(bf)
    CT = jnp.where(in_scores, st, alpha)
    KT = jnp.where(in_coupl, jnp.exp(CT), 0.0).astype(bf)

    rcol = lax.broadcasted_iota(jnp.int32, (PAD, 1), 0)
    mu = jnp.where(rcol < N, 1.0 / (2.0 * N),
                   jnp.where(rcol == N, 0.5, 0.0)).astype(f32)
    live = rcol <= N

    def body(_, uv):
        pu, pv = uv
        r = jnp.dot(K, pv.astype(bf), preferred_element_type=f32)
        pu = mu / jnp.where(live, r, 1.0)
        c = jnp.dot(KT, pu.astype(bf), preferred_element_type=f32)
        pv = mu / jnp.where(live, c, 1.0)
        return (pu, pv)

    pu0 = jnp.ones((PAD, 1), f32)
    pu, pv = lax.fori_loop(0, SINK_ITERS, body, (pu0, pu0))

    u = jnp.log(pu)
    vt = jnp.log(pv).reshape(1, PAD)
    out_ref[0] = C + u + vt + jnp.log(2.0 * N).astype(f32)


def kernel(quer_feat_lc, quer_feat_te, lc_W1, lc_b1, lc_W2, lc_b2,
           te_W1, te_b1, te_W2, te_b2, bin_score):
    B = quer_feat_lc.shape[0]
    pad_rows = PAD - quer_feat_lc.shape[1]
    xlc = jnp.pad(quer_feat_lc, ((0, 0), (0, pad_rows), (0, 0)))
    xte = jnp.pad(quer_feat_te, ((0, 0), (0, pad_rows), (0, 0)))
    alpha = jnp.reshape(bin_score, (1, 1)).astype(jnp.float32)

    d = D_MODEL
    mat = pl.BlockSpec((d, d), lambda b: (0, 0))
    vec = pl.BlockSpec((1, d), lambda b: (0, 0))

    out = pl.pallas_call(
        _ot_kernel,
        grid=(B,),
        in_specs=[
            pl.BlockSpec((1, PAD, d), lambda b: (b, 0, 0)),
            pl.BlockSpec((1, PAD, d), lambda b: (b, 0, 0)),
            mat, vec, mat, vec,
            mat, vec, mat, vec,
            pl.BlockSpec((1, 1), lambda b: (0, 0)),
        ],
        out_specs=pl.BlockSpec((1, PAD, PAD), lambda b: (b, 0, 0)),
        out_shape=jax.ShapeDtypeStruct((B, PAD, PAD), jnp.float32),
        compiler_params=pltpu.CompilerParams(
            dimension_semantics=("parallel",)),
    )(xlc, xte,
      lc_W1, lc_b1.reshape(1, d), lc_W2, lc_b2.reshape(1, d),
      te_W1, te_b1.reshape(1, d), te_W2, te_b2.reshape(1, d),
      alpha)
    return out[:, :N + 1, :N + 1]


# speedup vs baseline: 1.2717x; 1.2717x over previous
import jax
import jax.numpy as jnp
from jax import lax
from jax.experimental import pallas as pl
from jax.experimental.pallas import tpu as pltpu

D_MODEL = 256
SINK_ITERS = 50
N = 1000
PAD = 1024
BPG = 2


def _ot_kernel(xlc_ref, xte_ref, w1l_ref, b1l_ref, w2l_ref, b2l_ref,
               w1t_ref, b1t_ref, w2t_ref, b2t_ref, alpha_ref, out_ref):
    f32 = jnp.float32
    bf = jnp.bfloat16
    inv_sqrt_d = 1.0 / (D_MODEL ** 0.5)

    alpha = alpha_ref[0, 0]
    ri = lax.broadcasted_iota(jnp.int32, (PAD, PAD), 0)
    ci = lax.broadcasted_iota(jnp.int32, (PAD, PAD), 1)
    in_scores = (ri < N) & (ci < N)
    in_coupl = (ri <= N) & (ci <= N)

    rcol = lax.broadcasted_iota(jnp.int32, (PAD, 1), 0)
    mu = jnp.where(rcol < N, 1.0 / (2.0 * N),
                   jnp.where(rcol == N, 0.5, 0.0)).astype(f32)
    live = rcol <= N

    Cs, Ks, KTs = [], [], []
    for b in range(BPG):
        xlc = xlc_ref[0, b]
        xte = xte_ref[0, b]
        h = jnp.maximum(jnp.dot(xlc, w1l_ref[...], preferred_element_type=f32)
                        + b1l_ref[...], 0.0)
        f_lc = (jnp.dot(h, w2l_ref[...], preferred_element_type=f32)
                + b2l_ref[...]).astype(bf)
        h = jnp.maximum(jnp.dot(xte, w1t_ref[...], preferred_element_type=f32)
                        + b1t_ref[...], 0.0)
        f_te = (jnp.dot(h, w2t_ref[...], preferred_element_type=f32)
                + b2t_ref[...]).astype(bf)
        s = lax.dot_general(f_lc, f_te, (((1,), (1,)), ((), ())),
                            preferred_element_type=f32) * inv_sqrt_d
        st = lax.dot_general(f_te, f_lc, (((1,), (1,)), ((), ())),
                             preferred_element_type=f32) * inv_sqrt_d
        C = jnp.where(in_scores, s, alpha)
        CT = jnp.where(in_scores, st, alpha)
        Cs.append(C)
        Ks.append(jnp.where(in_coupl, jnp.exp(C), 0.0).astype(bf))
        KTs.append(jnp.where(in_coupl, jnp.exp(CT), 0.0).astype(bf))

    def body(_, uvs):
        out = []
        for b in range(BPG):
            pu, pv = uvs[b]
            r = jnp.dot(Ks[b], pv.astype(bf), preferred_element_type=f32)
            pu = mu / jnp.where(live, r, 1.0)
            c = jnp.dot(KTs[b], pu.astype(bf), preferred_element_type=f32)
            pv = mu / jnp.where(live, c, 1.0)
            out.append((pu, pv))
        return tuple(out)

    ones = jnp.ones((PAD, 1), f32)
    uvs = lax.fori_loop(0, SINK_ITERS, body,
                        tuple((ones, ones) for _ in range(BPG)))

    lognorm = jnp.log(2.0 * N).astype(f32)
    for b in range(BPG):
        pu, pv = uvs[b]
        u = jnp.log(pu)
        vt = jnp.log(pv).reshape(1, PAD)
        out_ref[0, b] = Cs[b] + u + vt + lognorm


def kernel(quer_feat_lc, quer_feat_te, lc_W1, lc_b1, lc_W2, lc_b2,
           te_W1, te_b1, te_W2, te_b2, bin_score):
    B = quer_feat_lc.shape[0]
    G = B // BPG
    pad_rows = PAD - quer_feat_lc.shape[1]
    xlc = jnp.pad(quer_feat_lc, ((0, 0), (0, pad_rows), (0, 0)))
    xte = jnp.pad(quer_feat_te, ((0, 0), (0, pad_rows), (0, 0)))
    xlc = xlc.reshape(G, BPG, PAD, D_MODEL)
    xte = xte.reshape(G, BPG, PAD, D_MODEL)
    alpha = jnp.reshape(bin_score, (1, 1)).astype(jnp.float32)

    d = D_MODEL
    mat = pl.BlockSpec((d, d), lambda g: (0, 0))
    vec = pl.BlockSpec((1, d), lambda g: (0, 0))

    out = pl.pallas_call(
        _ot_kernel,
        grid=(G,),
        in_specs=[
            pl.BlockSpec((1, BPG, PAD, d), lambda g: (g, 0, 0, 0)),
            pl.BlockSpec((1, BPG, PAD, d), lambda g: (g, 0, 0, 0)),
            mat, vec, mat, vec,
            mat, vec, mat, vec,
            pl.BlockSpec((1, 1), lambda g: (0, 0)),
        ],
        out_specs=pl.BlockSpec((1, BPG, PAD, PAD), lambda g: (g, 0, 0, 0)),
        out_shape=jax.ShapeDtypeStruct((G, BPG, PAD, PAD), jnp.float32),
        compiler_params=pltpu.CompilerParams(
            dimension_semantics=("parallel",)),
    )(xlc, xte,
      lc_W1, lc_b1.reshape(1, d), lc_W2, lc_b2.reshape(1, d),
      te_W1, te_b1.reshape(1, d), te_W2, te_b2.reshape(1, d),
      alpha)
    return out.reshape(B, PAD, PAD)[:, :N + 1, :N + 1]


# speedup vs baseline: 1.2724x; 1.0006x over previous
import jax
import jax.numpy as jnp
from jax import lax
from jax.experimental import pallas as pl
from jax.experimental.pallas import tpu as pltpu

D_MODEL = 256
SINK_ITERS = 50
N = 1000
PAD = 1024
BPG = 2


def _ot_kernel(xlc_ref, xte_ref, w1l_ref, b1l_ref, w2l_ref, b2l_ref,
               w1t_ref, b1t_ref, w2t_ref, b2t_ref, alpha_ref, out_ref):
    f32 = jnp.float32
    bf = jnp.bfloat16
    inv_sqrt_d = 1.0 / (D_MODEL ** 0.5)

    alpha = alpha_ref[0, 0]
    ri = lax.broadcasted_iota(jnp.int32, (PAD, PAD), 0)
    ci = lax.broadcasted_iota(jnp.int32, (PAD, PAD), 1)
    in_scores = (ri < N) & (ci < N)
    in_coupl = (ri <= N) & (ci <= N)

    rcol = lax.broadcasted_iota(jnp.int32, (PAD, 1), 0)
    mu = jnp.where(rcol < N, 1.0 / (2.0 * N),
                   jnp.where(rcol == N, 0.5, 0.0)).astype(f32)
    live = rcol <= N

    Ks, KTs = [], []
    for b in range(BPG):
        xlc = xlc_ref[0, b]
        xte = xte_ref[0, b]
        h = jnp.maximum(jnp.dot(xlc, w1l_ref[...], preferred_element_type=f32)
                        + b1l_ref[...], 0.0)
        f_lc = (jnp.dot(h, w2l_ref[...], preferred_element_type=f32)
                + b2l_ref[...]).astype(bf)
        h = jnp.maximum(jnp.dot(xte, w1t_ref[...], preferred_element_type=f32)
                        + b1t_ref[...], 0.0)
        f_te = (jnp.dot(h, w2t_ref[...], preferred_element_type=f32)
                + b2t_ref[...]).astype(bf)
        s = lax.dot_general(f_lc, f_te, (((1,), (1,)), ((), ())),
                            preferred_element_type=f32) * inv_sqrt_d
        st = lax.dot_general(f_te, f_lc, (((1,), (1,)), ((), ())),
                             preferred_element_type=f32) * inv_sqrt_d
        C = jnp.where(in_scores, s, alpha)
        CT = jnp.where(in_scores, st, alpha)
        out_ref[0, b] = C
        Ks.append(jnp.where(in_coupl, jnp.exp(C), 0.0).astype(bf))
        KTs.append(jnp.where(in_coupl, jnp.exp(CT), 0.0).astype(bf))

    def body(_, uvs):
        out = []
        for b in range(BPG):
            pu, pv = uvs[b]
            r = jnp.dot(Ks[b], pv.astype(bf), preferred_element_type=f32)
            pu = mu / jnp.where(live, r, 1.0)
            c = jnp.dot(KTs[b], pu.astype(bf), preferred_element_type=f32)
            pv = mu / jnp.where(live, c, 1.0)
            out.append((pu, pv))
        return tuple(out)

    ones = jnp.ones((PAD, 1), f32)
    uvs = lax.fori_loop(0, SINK_ITERS, body,
                        tuple((ones, ones) for _ in range(BPG)))

    lognorm = jnp.log(2.0 * N).astype(f32)
    for b in range(BPG):
        pu, pv = uvs[b]
        u = jnp.log(pu)
        vt = jnp.log(pv).reshape(1, PAD)
        out_ref[0, b] = out_ref[0, b] + u + vt + lognorm


def kernel(quer_feat_lc, quer_feat_te, lc_W1, lc_b1, lc_W2, lc_b2,
           te_W1, te_b1, te_W2, te_b2, bin_score):
    B = quer_feat_lc.shape[0]
    G = B // BPG
    pad_rows = PAD - quer_feat_lc.shape[1]
    xlc = jnp.pad(quer_feat_lc, ((0, 0), (0, pad_rows), (0, 0)))
    xte = jnp.pad(quer_feat_te, ((0, 0), (0, pad_rows), (0, 0)))
    xlc = xlc.reshape(G, BPG, PAD, D_MODEL)
    xte = xte.reshape(G, BPG, PAD, D_MODEL)
    alpha = jnp.reshape(bin_score, (1, 1)).astype(jnp.float32)

    d = D_MODEL
    mat = pl.BlockSpec((d, d), lambda g: (0, 0))
    vec = pl.BlockSpec((1, d), lambda g: (0, 0))

    out = pl.pallas_call(
        _ot_kernel,
        grid=(G,),
        in_specs=[
            pl.BlockSpec((1, BPG, PAD, d), lambda g: (g, 0, 0, 0)),
            pl.BlockSpec((1, BPG, PAD, d), lambda g: (g, 0, 0, 0)),
            mat, vec, mat, vec,
            mat, vec, mat, vec,
            pl.BlockSpec((1, 1), lambda g: (0, 0)),
        ],
        out_specs=pl.BlockSpec((1, BPG, PAD, PAD), lambda g: (g, 0, 0, 0)),
        out_shape=jax.ShapeDtypeStruct((G, BPG, PAD, PAD), jnp.float32),
        compiler_params=pltpu.CompilerParams(
            dimension_semantics=("parallel",)),
    )(xlc, xte,
      lc_W1, lc_b1.reshape(1, d), lc_W2, lc_b2.reshape(1, d),
      te_W1, te_b1.reshape(1, d), te_W2, te_b2.reshape(1, d),
      alpha)
    return out.reshape(B, PAD, PAD)[:, :N + 1, :N + 1]


# speedup vs baseline: 1.4749x; 1.1592x over previous
import jax
import jax.numpy as jnp
from jax import lax
from jax.experimental import pallas as pl
from jax.experimental.pallas import tpu as pltpu

D_MODEL = 256
SINK_ITERS = 50
N = 1000
PAD = 1024
BPG = 4


def _ot_kernel(xlc_ref, xte_ref, w1l_ref, b1l_ref, w2l_ref, b2l_ref,
               w1t_ref, b1t_ref, w2t_ref, b2t_ref, alpha_ref, out_ref):
    f32 = jnp.float32
    bf = jnp.bfloat16
    inv_sqrt_d = 1.0 / (D_MODEL ** 0.5)

    alpha = alpha_ref[0, 0]
    ri = lax.broadcasted_iota(jnp.int32, (PAD, PAD), 0)
    ci = lax.broadcasted_iota(jnp.int32, (PAD, PAD), 1)
    in_scores = (ri < N) & (ci < N)
    in_coupl = (ri <= N) & (ci <= N)

    rcol = lax.broadcasted_iota(jnp.int32, (PAD, 1), 0)
    mu = jnp.where(rcol < N, 1.0 / (2.0 * N),
                   jnp.where(rcol == N, 0.5, 0.0)).astype(f32)
    live = rcol <= N

    Ks = []
    for b in range(BPG):
        xlc = xlc_ref[0, b]
        xte = xte_ref[0, b]
        h = jnp.maximum(jnp.dot(xlc, w1l_ref[...], preferred_element_type=f32)
                        + b1l_ref[...], 0.0)
        f_lc = (jnp.dot(h, w2l_ref[...], preferred_element_type=f32)
                + b2l_ref[...]).astype(bf)
        h = jnp.maximum(jnp.dot(xte, w1t_ref[...], preferred_element_type=f32)
                        + b1t_ref[...], 0.0)
        f_te = (jnp.dot(h, w2t_ref[...], preferred_element_type=f32)
                + b2t_ref[...]).astype(bf)
        s = lax.dot_general(f_lc, f_te, (((1,), (1,)), ((), ())),
                            preferred_element_type=f32) * inv_sqrt_d
        C = jnp.where(in_scores, s, alpha)
        out_ref[0, b] = C
        Ks.append(jnp.where(in_coupl, jnp.exp(C), 0.0).astype(bf))

    def body(_, uvs):
        out = []
        for b in range(BPG):
            pu, pv = uvs[b]
            r = jnp.dot(Ks[b], pv.astype(bf), preferred_element_type=f32)
            pu = mu / jnp.where(live, r, 1.0)
            c = lax.dot_general(Ks[b], pu.astype(bf), (((0,), (0,)), ((), ())),
                                preferred_element_type=f32)
            pv = mu / jnp.where(live, c, 1.0)
            out.append((pu, pv))
        return tuple(out)

    ones = jnp.ones((PAD, 1), f32)
    uvs = lax.fori_loop(0, SINK_ITERS, body,
                        tuple((ones, ones) for _ in range(BPG)))

    lognorm = jnp.log(2.0 * N).astype(f32)
    for b in range(BPG):
        pu, pv = uvs[b]
        u = jnp.log(pu)
        vt = jnp.log(pv).reshape(1, PAD)
        out_ref[0, b] = out_ref[0, b] + u + vt + lognorm


def kernel(quer_feat_lc, quer_feat_te, lc_W1, lc_b1, lc_W2, lc_b2,
           te_W1, te_b1, te_W2, te_b2, bin_score):
    B = quer_feat_lc.shape[0]
    G = B // BPG
    pad_rows = PAD - quer_feat_lc.shape[1]
    xlc = jnp.pad(quer_feat_lc, ((0, 0), (0, pad_rows), (0, 0)))
    xte = jnp.pad(quer_feat_te, ((0, 0), (0, pad_rows), (0, 0)))
    xlc = xlc.reshape(G, BPG, PAD, D_MODEL)
    xte = xte.reshape(G, BPG, PAD, D_MODEL)
    alpha = jnp.reshape(bin_score, (1, 1)).astype(jnp.float32)

    d = D_MODEL
    mat = pl.BlockSpec((d, d), lambda g: (0, 0))
    vec = pl.BlockSpec((1, d), lambda g: (0, 0))

    out = pl.pallas_call(
        _ot_kernel,
        grid=(G,),
        in_specs=[
            pl.BlockSpec((1, BPG, PAD, d), lambda g: (g, 0, 0, 0)),
            pl.BlockSpec((1, BPG, PAD, d), lambda g: (g, 0, 0, 0)),
            mat, vec, mat, vec,
            mat, vec, mat, vec,
            pl.BlockSpec((1, 1), lambda g: (0, 0)),
        ],
        out_specs=pl.BlockSpec((1, BPG, PAD, PAD), lambda g: (g, 0, 0, 0)),
        out_shape=jax.ShapeDtypeStruct((G, BPG, PAD, PAD), jnp.float32),
        compiler_params=pltpu.CompilerParams(
            dimension_semantics=("parallel",)),
    )(xlc, xte,
      lc_W1, lc_b1.reshape(1, d), lc_W2, lc_b2.reshape(1, d),
      te_W1, te_b1.reshape(1, d), te_W2, te_b2.reshape(1, d),
      alpha)
    return out.reshape(B, PAD, PAD)[:, :N + 1, :N + 1]


# speedup vs baseline: 1.6196x; 1.0981x over previous
import jax
import jax.numpy as jnp
from jax import lax
from jax.experimental import pallas as pl
from jax.experimental.pallas import tpu as pltpu

D_MODEL = 256
SINK_ITERS = 50
N = 1000
PAD = 1024
NOUT = N + 1


def _ot_kernel(xlc_ref, xte_ref, w1l_ref, b1l_ref, w2l_ref, b2l_ref,
               w1t_ref, b1t_ref, w2t_ref, b2t_ref, alpha_ref, out_ref):
    f32 = jnp.float32
    bf = jnp.bfloat16
    inv_sqrt_d = 1.0 / (D_MODEL ** 0.5)
    B = out_ref.shape[0]

    alpha = alpha_ref[0, 0]
    ri = lax.broadcasted_iota(jnp.int32, (PAD, PAD), 0)
    ci = lax.broadcasted_iota(jnp.int32, (PAD, PAD), 1)
    in_scores = (ri < N) & (ci < N)
    in_coupl = (ri <= N) & (ci <= N)

    rcol = lax.broadcasted_iota(jnp.int32, (PAD, 1), 0)
    mu = jnp.where(rcol < N, 1.0 / (2.0 * N),
                   jnp.where(rcol == N, 0.5, 0.0)).astype(f32)
    live = rcol <= N

    Ks = []
    for b in range(B):
        xlc = xlc_ref[b]
        xte = xte_ref[b]
        h = jnp.maximum(jnp.dot(xlc, w1l_ref[...], preferred_element_type=f32)
                        + b1l_ref[...], 0.0)
        f_lc = (jnp.dot(h, w2l_ref[...], preferred_element_type=f32)
                + b2l_ref[...]).astype(bf)
        h = jnp.maximum(jnp.dot(xte, w1t_ref[...], preferred_element_type=f32)
                        + b1t_ref[...], 0.0)
        f_te = (jnp.dot(h, w2t_ref[...], preferred_element_type=f32)
                + b2t_ref[...]).astype(bf)
        s = lax.dot_general(f_lc, f_te, (((1,), (1,)), ((), ())),
                            preferred_element_type=f32) * inv_sqrt_d
        s = jnp.pad(s, ((0, PAD - N), (0, PAD - N)))
        C = jnp.where(in_scores, s, alpha)
        out_ref[b] = C[:NOUT, :NOUT]
        Ks.append(jnp.where(in_coupl, jnp.exp(C), 0.0).astype(bf))

    def body(_, uvs):
        out = []
        for b in range(B):
            pu, pv = uvs[b]
            r = jnp.dot(Ks[b], pv.astype(bf), preferred_element_type=f32)
            pu = mu / jnp.where(live, r, 1.0)
            c = lax.dot_general(Ks[b], pu.astype(bf), (((0,), (0,)), ((), ())),
                                preferred_element_type=f32)
            pv = mu / jnp.where(live, c, 1.0)
            out.append((pu, pv))
        return tuple(out)

    ones = jnp.ones((PAD, 1), f32)
    uvs = lax.fori_loop(0, SINK_ITERS, body,
                        tuple((ones, ones) for _ in range(B)))

    lognorm = jnp.log(2.0 * N).astype(f32)
    for b in range(B):
        pu, pv = uvs[b]
        u = jnp.log(pu)[:NOUT]
        vt = jnp.log(pv).reshape(1, PAD)[:, :NOUT]
        out_ref[b] = out_ref[b] + u + vt + lognorm


def kernel(quer_feat_lc, quer_feat_te, lc_W1, lc_b1, lc_W2, lc_b2,
           te_W1, te_b1, te_W2, te_b2, bin_score):
    B = quer_feat_lc.shape[0]
    alpha = jnp.reshape(bin_score, (1, 1)).astype(jnp.float32)

    d = D_MODEL
    mat = pl.BlockSpec((d, d), lambda: (0, 0))
    vec = pl.BlockSpec((1, d), lambda: (0, 0))

    return pl.pallas_call(
        _ot_kernel,
        in_specs=[
            pl.BlockSpec((B, N, d), lambda: (0, 0, 0)),
            pl.BlockSpec((B, N, d), lambda: (0, 0, 0)),
            mat, vec, mat, vec,
            mat, vec, mat, vec,
            pl.BlockSpec((1, 1), lambda: (0, 0)),
        ],
        out_specs=pl.BlockSpec((B, NOUT, NOUT), lambda: (0, 0, 0)),
        out_shape=jax.ShapeDtypeStruct((B, NOUT, NOUT), jnp.float32),
    )(quer_feat_lc, quer_feat_te,
      lc_W1, lc_b1.reshape(1, d), lc_W2, lc_b2.reshape(1, d),
      te_W1, te_b1.reshape(1, d), te_W2, te_b2.reshape(1, d),
      alpha)


# speedup vs baseline: 3.3945x; 2.0959x over previous
import jax
import jax.numpy as jnp
from jax import lax
from jax.experimental import pallas as pl
from jax.experimental.pallas import tpu as pltpu

D_MODEL = 256
SINK_ITERS = 50
N = 1000
PAD = 1024
NOUT = N + 1


def _ot_kernel(xlc_ref, xte_ref, w1l_ref, b1l_ref, w2l_ref, b2l_ref,
               w1t_ref, b1t_ref, w2t_ref, b2t_ref, alpha_ref, out_ref):
    f32 = jnp.float32
    bf = jnp.bfloat16
    inv_sqrt_d = 1.0 / (D_MODEL ** 0.5)
    B = out_ref.shape[0]

    alpha = alpha_ref[0, 0]
    ri = lax.broadcasted_iota(jnp.int32, (PAD, PAD), 0)
    ci = lax.broadcasted_iota(jnp.int32, (PAD, PAD), 1)
    in_scores = (ri < N) & (ci < N)
    in_coupl = (ri <= N) & (ci <= N)

    crow = lax.broadcasted_iota(jnp.int32, (1, PAD), 1)
    mu = jnp.where(crow < N, 1.0 / (2.0 * N),
                   jnp.where(crow == N, 0.5, 0.0)).astype(f32)
    live = crow <= N

    Ks = []
    KTs = []
    for b in range(B):
        xlc = xlc_ref[b]
        xte = xte_ref[b]
        h = jnp.maximum(jnp.dot(xlc, w1l_ref[...], preferred_element_type=f32)
                        + b1l_ref[...], 0.0)
        f_lc = (jnp.dot(h, w2l_ref[...], preferred_element_type=f32)
                + b2l_ref[...]).astype(bf)
        h = jnp.maximum(jnp.dot(xte, w1t_ref[...], preferred_element_type=f32)
                        + b1t_ref[...], 0.0)
        f_te = (jnp.dot(h, w2t_ref[...], preferred_element_type=f32)
                + b2t_ref[...]).astype(bf)
        s = lax.dot_general(f_lc, f_te, (((1,), (1,)), ((), ())),
                            preferred_element_type=f32) * inv_sqrt_d
        s = jnp.pad(s, ((0, PAD - N), (0, PAD - N)))
        C = jnp.where(in_scores, s, alpha)
        out_ref[b] = C[:NOUT, :NOUT]
        Ks.append(jnp.where(in_coupl, jnp.exp(C), 0.0).astype(bf))
        st = lax.dot_general(f_te, f_lc, (((1,), (1,)), ((), ())),
                             preferred_element_type=f32) * inv_sqrt_d
        st = jnp.pad(st, ((0, PAD - N), (0, PAD - N)))
        Ct = jnp.where(in_scores, st, alpha)
        KTs.append(jnp.where(in_coupl, jnp.exp(Ct), 0.0).astype(bf))

    def body(_, uvs):
        out = []
        for b in range(B):
            pu, pv = uvs[b]
            r = jnp.dot(pv.astype(bf), KTs[b], preferred_element_type=f32)
            pu = mu / jnp.where(live, r, 1.0)
            c = jnp.dot(pu.astype(bf), Ks[b], preferred_element_type=f32)
            pv = mu / jnp.where(live, c, 1.0)
            out.append((pu, pv))
        return tuple(out)

    ones = jnp.ones((1, PAD), f32)
    uvs = lax.fori_loop(0, SINK_ITERS, body,
                        tuple((ones, ones) for _ in range(B)))

    lognorm = jnp.log(2.0 * N).astype(f32)
    for b in range(B):
        pu, pv = uvs[b]
        u = jnp.log(pu).reshape(PAD, 1)[:NOUT]
        vt = jnp.log(pv)[:, :NOUT]
        out_ref[b] = out_ref[b] + u + vt + lognorm


def kernel(quer_feat_lc, quer_feat_te, lc_W1, lc_b1, lc_W2, lc_b2,
           te_W1, te_b1, te_W2, te_b2, bin_score):
    B = quer_feat_lc.shape[0]
    alpha = jnp.reshape(bin_score, (1, 1)).astype(jnp.float32)

    d = D_MODEL
    mat = pl.BlockSpec((d, d), lambda: (0, 0))
    vec = pl.BlockSpec((1, d), lambda: (0, 0))

    return pl.pallas_call(
        _ot_kernel,
        in_specs=[
            pl.BlockSpec((B, N, d), lambda: (0, 0, 0)),
            pl.BlockSpec((B, N, d), lambda: (0, 0, 0)),
            mat, vec, mat, vec,
            mat, vec, mat, vec,
            pl.BlockSpec((1, 1), lambda: (0, 0)),
        ],
        out_specs=pl.BlockSpec((B, NOUT, NOUT), lambda: (0, 0, 0)),
        out_shape=jax.ShapeDtypeStruct((B, NOUT, NOUT), jnp.float32),
    )(quer_feat_lc, quer_feat_te,
      lc_W1, lc_b1.reshape(1, d), lc_W2, lc_b2.reshape(1, d),
      te_W1, te_b1.reshape(1, d), te_W2, te_b2.reshape(1, d),
      alpha)
